# Initial kernel scaffold; baseline (speedup 1.0000x reference)
import functools

import jax
import jax.numpy as jnp
from jax import lax
from jax.experimental import pallas as pl
from jax.experimental.pallas import tpu as pltpu
from jax.experimental.pallas import tpu_sc as plsc

_INFO = plsc.get_sparse_core_info()
_NW = _INFO.num_cores * _INFO.num_subcores


def kernel(x, weight):
    B, S = x.shape
    V, D = weight.shape
    N = B * S
    x_flat = x.reshape(N)

    n_per_w = N // _NW
    chunk = min(2048, n_per_w)
    n_chunks = n_per_w // chunk

    mesh = plsc.VectorSubcoreMesh(core_axis_name="c", subcore_axis_name="s")

    @functools.partial(
        pl.kernel,
        out_type=jax.ShapeDtypeStruct((N, D), jnp.float32),
        mesh=mesh,
        scratch_types=[
            pltpu.VMEM((chunk,), jnp.int32),
            pltpu.VMEM((chunk, D), jnp.float32),
            pltpu.SemaphoreType.DMA,
        ],
    )
    def emb(x_hbm, w_hbm, out_hbm, idx_v, rows_v, sem):
        wid = lax.axis_index("s") * _INFO.num_cores + lax.axis_index("c")
        base0 = wid * n_per_w

        def body(i, carry):
            base = base0 + i * chunk
            pltpu.sync_copy(x_hbm.at[pl.ds(base, chunk)], idx_v)
            pltpu.async_copy(w_hbm.at[idx_v], rows_v, sem).wait()
            pltpu.sync_copy(rows_v, out_hbm.at[pl.ds(base, chunk)])
            return carry

        lax.fori_loop(0, n_chunks, body, 0)

    out = emb(x_flat, weight)
    return out.reshape(B, S, D)


# speedup vs baseline: 17.7514x; 17.7514x over previous
import functools

import jax
import jax.numpy as jnp
from jax import lax
from jax.experimental import pallas as pl
from jax.experimental.pallas import tpu as pltpu
from jax.experimental.pallas import tpu_sc as plsc

try:
    _INFO = plsc.get_sparse_core_info()
    _NC, _NS = _INFO.num_cores, _INFO.num_subcores
except Exception:
    _NC, _NS = 2, 16
_NW = _NC * _NS

_L = 16
_TJ = 8
_TI = 128


def kernel(x, weight):
    B, S = x.shape
    V, D = weight.shape
    N = B * S
    njb = S // _TJ
    nib = B // _TI
    nu = njb * nib
    u_per_w = nu // _NW

    tab_size = V * D + (-(V * D)) % 8
    wflat = jnp.concatenate(
        [weight.reshape(-1), jnp.zeros((tab_size - V * D,), jnp.float32)]
    )

    mesh = plsc.VectorSubcoreMesh(
        core_axis_name="c", subcore_axis_name="s", num_cores=_NC, num_subcores=_NS
    )

    @functools.partial(
        pl.kernel,
        out_type=jax.ShapeDtypeStruct((N * D,), jnp.float32),
        mesh=mesh,
        scratch_types=[
            pltpu.VMEM((tab_size,), jnp.float32),
            pltpu.VMEM((_TI, _TJ), jnp.int32),
            pltpu.VMEM((D * _TJ * _TI,), jnp.float32),
            pltpu.SemaphoreType.DMA,
        ],
        compiler_params=pltpu.CompilerParams(
            use_tc_tiling_on_sc=False, needs_layout_passes=False
        ),
    )
    def emb(x_hbm, w_hbm, out_hbm, tab_v, xblk_v, out_v, sem):
        wid = lax.axis_index("s") * _NC + lax.axis_index("c")
        u0 = wid * u_per_w
        pltpu.sync_copy(w_hbm, tab_v)
        iota = lax.iota(jnp.int32, _L)
        blk = _TJ * _TI

        def unit_body(u, carry):
            jb = u // nib
            ib = u % nib
            pltpu.sync_copy(
                x_hbm.at[pl.ds(ib * _TI, _TI), pl.ds(jb * _TJ, _TJ)], xblk_v
            )

            def k_body(k, carry2):
                jr = k // 8
                kk = k % 8
                idxv = plsc.load_gather(
                    xblk_v, [iota + kk * _L, jnp.broadcast_to(jr, (_L,))]
                )
                a = idxv * D
                for d in range(D):
                    out_v[pl.ds(d * blk + k * _L, _L)] = plsc.load_gather(
                        tab_v, [a + d]
                    )
                return carry2

            lax.fori_loop(0, _TJ * _TI // _L, k_body, 0)

            copies = [
                pltpu.async_copy(
                    out_v.at[pl.ds(d * blk, blk)],
                    out_hbm.at[pl.ds(d * (nu * blk) + u * blk, blk)],
                    sem,
                )
                for d in range(D)
            ]
            for c in copies:
                c.wait()
            return carry

        lax.fori_loop(u0, u0 + u_per_w, unit_body, 0)

    out = emb(x, wflat)
    return (
        out.reshape(D, njb, nib, _TJ, _TI)
        .transpose(2, 4, 1, 3, 0)
        .reshape(B, S, D)
    )


# speedup vs baseline: 22.5263x; 1.2690x over previous
import functools

import jax
import jax.numpy as jnp
from jax import lax
from jax.experimental import pallas as pl
from jax.experimental.pallas import tpu as pltpu
from jax.experimental.pallas import tpu_sc as plsc

try:
    _INFO = plsc.get_sparse_core_info()
    _NC, _NS = _INFO.num_cores, _INFO.num_subcores
except Exception:
    _NC, _NS = 2, 16
_NW = _NC * _NS

_L = 16
_TJ = 8
_TI = 128
_KP = 2


def kernel(x, weight):
    B, S = x.shape
    V, D = weight.shape
    N = B * S
    njb = S // _TJ
    nib = B // _TI
    nu = njb * nib
    ib_per_w = nib // _NW
    n_units = njb * (ib_per_w // _KP)
    blk = _TJ * _TI
    piece = _KP * blk
    obuf = D * piece

    tab_size = V * D + (-(V * D)) % 8
    wflat = jnp.concatenate(
        [weight.reshape(-1), jnp.zeros((tab_size - V * D,), jnp.float32)]
    )

    mesh = plsc.VectorSubcoreMesh(
        core_axis_name="c", subcore_axis_name="s", num_cores=_NC, num_subcores=_NS
    )

    @functools.partial(
        pl.kernel,
        out_type=jax.ShapeDtypeStruct((N * D,), jnp.float32),
        mesh=mesh,
        scratch_types=[
            pltpu.VMEM((tab_size,), jnp.float32),
            pltpu.VMEM((_KP * _TI, _TJ), jnp.int32),
            pltpu.VMEM((_KP * _TI, _TJ), jnp.int32),
            pltpu.VMEM((obuf,), jnp.float32),
            pltpu.VMEM((obuf,), jnp.float32),
            pltpu.SemaphoreType.DMA,
            pltpu.SemaphoreType.DMA,
            pltpu.SemaphoreType.DMA,
            pltpu.SemaphoreType.DMA,
        ],
        compiler_params=pltpu.CompilerParams(
            use_tc_tiling_on_sc=False, needs_layout_passes=False
        ),
    )
    def emb(x_hbm, w_hbm, out_hbm, tab_v, xa, xb, oa, ob, sia, sib, soa, sob):
        wid = lax.axis_index("s") * _NC + lax.axis_index("c")
        ib0w = wid * ib_per_w
        pltpu.sync_copy(w_hbm, tab_v)
        iota = lax.iota(jnp.int32, _L)
        xs, os = [xa, xb], [oa, ob]
        sis, sos = [sia, sib], [soa, sob]

        def idx_src(n):
            jb = n // _KP
            i_lo = (ib0w + (n % _KP) * _KP) * _TI
            return x_hbm.at[pl.ds(i_lo, _KP * _TI), pl.ds(jb * _TJ, _TJ)]

        def fire_out(n, b):
            jb = n // _KP
            u = jb * nib + ib0w + (n % _KP) * _KP
            for d in range(D):
                pltpu.async_copy(
                    os[b].at[pl.ds(d * piece, piece)],
                    out_hbm.at[pl.ds(d * (nu * blk) + u * blk, piece)],
                    sos[b],
                )

        def compute(b):
            def k_body(k, carry):
                ibs_kk = k % (_KP * _TJ)
                jr = k // (_KP * _TJ)
                row0 = (ibs_kk // _TJ) * _TI + (ibs_kk % _TJ) * _L
                off0 = (ibs_kk // _TJ) * blk + jr * _TI + (ibs_kk % _TJ) * _L
                idxv = plsc.load_gather(
                    xs[b], [iota + row0, jnp.broadcast_to(jr, (_L,))]
                )
                a = idxv * D
                for d in range(D):
                    os[b][pl.ds(d * piece + off0, _L)] = plsc.load_gather(
                        tab_v, [a + d]
                    )
                return carry

            lax.fori_loop(0, _KP * blk // _L, k_body, 0)

        for b in range(2):
            pltpu.async_copy(idx_src(b), xs[b], sis[b])

        def g_body(g, carry):
            for b in range(2):
                n = _KP * g + b
                pltpu.make_async_copy(idx_src(n), xs[b], sis[b]).wait()
                @pl.when(g >= 1)
                def _():
                    pltpu.make_async_copy(
                        out_hbm.at[pl.ds(0, obuf)], os[b], sos[b]
                    ).wait()

                compute(b)
                fire_out(n, b)
                n2 = jnp.minimum(n + _KP, n_units - 1)
                pltpu.async_copy(idx_src(n2), xs[b], sis[b])
            return carry

        lax.fori_loop(0, n_units // _KP, g_body, 0)

        for b in range(2):
            pltpu.make_async_copy(idx_src(0), xs[b], sis[b]).wait()
            pltpu.make_async_copy(
                out_hbm.at[pl.ds(0, obuf)], os[b], sos[b]
            ).wait()

    out = emb(x, wflat)
    return (
        out.reshape(D, njb, nib, _TJ, _TI)
        .transpose(2, 4, 1, 3, 0)
        .reshape(B, S, D)
    )
